# baseline (speedup 1.0000x reference)
import jax
import jax.numpy as jnp
from jax.experimental import pallas as pl
from jax.experimental.pallas import tpu as pltpu

_B, _J, _H, _W = 128, 17, 64, 48
_HW = _H * _W
_TOPK = 8
_BB = 16


def _ohkm_kernel(o_ref, t_ref, w_ref, out_ref):
    d = o_ref[...] - t_ref[...]
    s = jnp.sum(d * d, axis=(2, 3))
    w = w_ref[...]
    vals = s * (w * w) * (0.5 / _HW)
    col = jax.lax.broadcasted_iota(jnp.int32, vals.shape, 1)
    acc = jnp.zeros((vals.shape[0],), jnp.float32)
    neg_inf = jnp.float32(-jnp.inf)
    for _ in range(_TOPK):
        m = jnp.max(vals, axis=1)
        acc = acc + m
        is_max = vals == m[:, None]
        first_idx = jnp.min(jnp.where(is_max, col, _J), axis=1)
        vals = jnp.where(col == first_idx[:, None], neg_inf, vals)
    partial = jnp.sum(acc) * (1.0 / (_TOPK * _B))

    @pl.when(pl.program_id(0) == 0)
    def _():
        out_ref[0, 0] = 0.0

    out_ref[0, 0] += partial


def kernel(outs, targets, target_weights):
    w = target_weights.reshape(_B, _J)
    out = pl.pallas_call(
        _ohkm_kernel,
        grid=(_B // _BB,),
        in_specs=[
            pl.BlockSpec((_BB, _J, _H, _W), lambda i: (i, 0, 0, 0)),
            pl.BlockSpec((_BB, _J, _H, _W), lambda i: (i, 0, 0, 0)),
            pl.BlockSpec((_BB, _J), lambda i: (i, 0)),
        ],
        out_specs=pl.BlockSpec(
            (1, 1), lambda i: (0, 0), memory_space=pltpu.SMEM
        ),
        out_shape=jax.ShapeDtypeStruct((1, 1), jnp.float32),
    )(outs, targets, w)
    return out.reshape(())


# speedup vs baseline: 2.3811x; 2.3811x over previous
import jax
import jax.numpy as jnp
from jax.experimental import pallas as pl
from jax.experimental.pallas import tpu as pltpu

_B, _J, _H, _W = 128, 17, 64, 48
_HW = _H * _W
_ROWS = _J * _HW // 128
_RPJ = _HW // 128
_TOPK = 8
_NB = 16
_GRID = _B // _NB


def _ohkm_kernel(o_ref, t_ref, w_ref, out_ref, s_ref):
    i = pl.program_id(0)
    d = o_ref[...] - t_ref[...]
    d2 = d * d
    s = jnp.sum(d2.reshape(_NB, _J, _RPJ, 128), axis=(2, 3))
    s_ref[pl.ds(i * _NB, _NB), :] = s

    @pl.when(i == _GRID - 1)
    def _():
        w = w_ref[...]
        vals = s_ref[...] * (w * w) * (0.5 / _HW)
        col = jax.lax.broadcasted_iota(jnp.int32, vals.shape, 1)
        acc = jnp.zeros((vals.shape[0],), jnp.float32)
        neg_inf = jnp.float32(-jnp.inf)
        for _ in range(_TOPK):
            m = jnp.max(vals, axis=1)
            acc = acc + m
            is_max = vals == m[:, None]
            first_idx = jnp.min(jnp.where(is_max, col, _J), axis=1)
            vals = jnp.where(col == first_idx[:, None], neg_inf, vals)
        out_ref[0, 0] = jnp.sum(acc) * (1.0 / (_TOPK * _B))


def kernel(outs, targets, target_weights):
    o = outs.reshape(_B, _ROWS, 128)
    t = targets.reshape(_B, _ROWS, 128)
    w = target_weights.reshape(_B, _J)
    out = pl.pallas_call(
        _ohkm_kernel,
        grid=(_GRID,),
        in_specs=[
            pl.BlockSpec((_NB, _ROWS, 128), lambda i: (i, 0, 0)),
            pl.BlockSpec((_NB, _ROWS, 128), lambda i: (i, 0, 0)),
            pl.BlockSpec((_B, _J), lambda i: (0, 0)),
        ],
        out_specs=pl.BlockSpec(
            (1, 1), lambda i: (0, 0), memory_space=pltpu.SMEM
        ),
        out_shape=jax.ShapeDtypeStruct((1, 1), jnp.float32),
        scratch_shapes=[pltpu.VMEM((_B, _J), jnp.float32)],
    )(o, t, w)
    return out.reshape(())


# speedup vs baseline: 7.3680x; 3.0944x over previous
import jax
import jax.numpy as jnp
from jax.experimental import pallas as pl
from jax.experimental.pallas import tpu as pltpu

_B, _J, _H, _W = 128, 17, 64, 48
_HW = _H * _W
_RB = 192
_GRID = _HW // _RB
_TOPK = 8


def _ohkm_kernel(o_ref, t_ref, w_ref, out_ref, s_ref):
    i = pl.program_id(0)
    d = o_ref[...] - t_ref[...]
    part = jnp.sum(d * d, axis=1)

    @pl.when(i == 0)
    def _():
        s_ref[...] = jnp.zeros((_J, _B), jnp.float32)

    s_ref[...] += part

    @pl.when(i == _GRID - 1)
    def _():
        w = w_ref[...]
        vals = s_ref[...] * (w * w) * (0.5 / _HW)
        row = jax.lax.broadcasted_iota(jnp.int32, vals.shape, 0)
        acc = jnp.zeros((_B,), jnp.float32)
        neg_inf = jnp.float32(-jnp.inf)
        for _ in range(_TOPK):
            m = jnp.max(vals, axis=0)
            acc = acc + m
            is_max = vals == m[None, :]
            first_idx = jnp.min(jnp.where(is_max, row, _J), axis=0)
            vals = jnp.where(row == first_idx[None, :], neg_inf, vals)
        out_ref[0, 0] = jnp.sum(acc) * (1.0 / (_TOPK * _B))


def kernel(outs, targets, target_weights):
    o = jnp.transpose(outs, (1, 2, 3, 0)).reshape(_J, _HW, _B)
    t = jnp.transpose(targets, (1, 2, 3, 0)).reshape(_J, _HW, _B)
    w = jnp.transpose(target_weights, (1, 2, 0)).reshape(_J, _B)
    out = pl.pallas_call(
        _ohkm_kernel,
        grid=(_GRID,),
        in_specs=[
            pl.BlockSpec((_J, _RB, _B), lambda i: (0, i, 0)),
            pl.BlockSpec((_J, _RB, _B), lambda i: (0, i, 0)),
            pl.BlockSpec((_J, _B), lambda i: (0, 0)),
        ],
        out_specs=pl.BlockSpec(
            (1, 1), lambda i: (0, 0), memory_space=pltpu.SMEM
        ),
        out_shape=jax.ShapeDtypeStruct((1, 1), jnp.float32),
        scratch_shapes=[pltpu.VMEM((_J, _B), jnp.float32)],
    )(o, t, w)
    return out.reshape(())


# speedup vs baseline: 8.5135x; 1.1555x over previous
import jax
import jax.numpy as jnp
from jax.experimental import pallas as pl
from jax.experimental.pallas import tpu as pltpu

_B, _J, _H, _W = 128, 17, 64, 48
_HW = _H * _W
_RB = 384
_GRID = _HW // _RB
_TOPK = 8


def _ohkm_kernel(o_ref, t_ref, w_ref, out_ref, s_ref):
    i = pl.program_id(0)
    d = o_ref[...] - t_ref[...]
    part = jnp.sum(d * d, axis=1)

    @pl.when(i == 0)
    def _():
        s_ref[...] = jnp.zeros((_J, _B), jnp.float32)

    s_ref[...] += part

    @pl.when(i == _GRID - 1)
    def _():
        w = w_ref[...]
        vals = s_ref[...] * (w * w) * (0.5 / _HW)
        row = jax.lax.broadcasted_iota(jnp.int32, vals.shape, 0)
        acc = jnp.zeros((_B,), jnp.float32)
        neg_inf = jnp.float32(-jnp.inf)
        for _ in range(_TOPK):
            m = jnp.max(vals, axis=0)
            acc = acc + m
            is_max = vals == m[None, :]
            first_idx = jnp.min(jnp.where(is_max, row, _J), axis=0)
            vals = jnp.where(row == first_idx[None, :], neg_inf, vals)
        out_ref[0, 0] = jnp.sum(acc) * (1.0 / (_TOPK * _B))


def kernel(outs, targets, target_weights):
    o = jnp.transpose(outs, (1, 2, 3, 0)).reshape(_J, _HW, _B)
    t = jnp.transpose(targets, (1, 2, 3, 0)).reshape(_J, _HW, _B)
    w = jnp.transpose(target_weights, (1, 2, 0)).reshape(_J, _B)
    out = pl.pallas_call(
        _ohkm_kernel,
        grid=(_GRID,),
        in_specs=[
            pl.BlockSpec((_J, _RB, _B), lambda i: (0, i, 0)),
            pl.BlockSpec((_J, _RB, _B), lambda i: (0, i, 0)),
            pl.BlockSpec((_J, _B), lambda i: (0, 0)),
        ],
        out_specs=pl.BlockSpec(
            (1, 1), lambda i: (0, 0), memory_space=pltpu.SMEM
        ),
        out_shape=jax.ShapeDtypeStruct((1, 1), jnp.float32),
        scratch_shapes=[pltpu.VMEM((_J, _B), jnp.float32)],
    )(o, t, w)
    return out.reshape(())
